# baseline (speedup 1.0000x reference)
import functools

import jax
import jax.numpy as jnp
from jax import lax
from jax.experimental import pallas as pl
from jax.experimental.pallas import tpu as pltpu
from jax.experimental.pallas import tpu_sc as plsc

_BATCH = 4096
_SEQ = 50
_EMB = 32
_SEQP = 52
_TCOL = 13
_TOTAL = _BATCH * _SEQP

_NC = 2
_NS = 16
_NW = _NC * _NS
_ROWS_PER_W = _TOTAL // _NW
_IDX_MINOR = 128
_GRP_PER_W = _ROWS_PER_W // _IDX_MINOR
_GRP_PER_CHUNK = 13
_N_CHUNKS = _GRP_PER_W // _GRP_PER_CHUNK
_CHUNK_ROWS = _GRP_PER_CHUNK * _IDX_MINOR


def _sc_gather(idx3d, table):
    mesh = plsc.VectorSubcoreMesh(core_axis_name="c", subcore_axis_name="s")

    @functools.partial(
        pl.kernel,
        mesh=mesh,
        out_type=jax.ShapeDtypeStruct((_TOTAL, _EMB), jnp.float32),
        scratch_types=[
            pltpu.VMEM((_GRP_PER_W, _IDX_MINOR), jnp.int32),
            pltpu.VMEM((_CHUNK_ROWS, _EMB), jnp.float32),
            pltpu.SemaphoreType.DMA,
        ],
        compiler_params=pltpu.CompilerParams(use_tc_tiling_on_sc=False),
    )
    def gather_kernel(idx_hbm, table_hbm, out_hbm, idx_v, rows_v, sem):
        wid = lax.axis_index("s") * _NC + lax.axis_index("c")
        row_base = wid * _ROWS_PER_W
        pltpu.sync_copy(idx_hbm.at[wid], idx_v)
        for c in range(_N_CHUNKS):
            copies = []
            for j in range(_GRP_PER_CHUNK):
                copies.append(pltpu.async_copy(
                    table_hbm.at[idx_v.at[c * _GRP_PER_CHUNK + j]],
                    rows_v.at[pl.ds(j * _IDX_MINOR, _IDX_MINOR)],
                    sem,
                ))
            for cp in copies:
                cp.wait()
            pltpu.sync_copy(
                rows_v,
                out_hbm.at[pl.ds(row_base + c * _CHUNK_ROWS, _CHUNK_ROWS)],
            )

    return gather_kernel(idx3d, table)


def _mlp(x4, W1p, b1, W2, b2):
    gblk = 64
    rows = gblk * 8

    def body(x_ref, w1_ref, b1_ref, w2_ref, b2_ref, o_ref):
        acc = None
        for t in range(_TCOL):
            xt = x_ref[:, t].reshape(rows, _IDX_MINOR)
            p = jnp.dot(xt, w1_ref[t], preferred_element_type=jnp.float32)
            acc = p if acc is None else acc + p
        h = jnp.maximum(acc + b1_ref[...], 0.0)
        o = jnp.dot(h, w2_ref[...], preferred_element_type=jnp.float32)
        o_ref[...] = jnp.maximum(o + b2_ref[...], 0.0)

    return pl.pallas_call(
        body,
        grid=(_BATCH // rows,),
        in_specs=[
            pl.BlockSpec((gblk, _TCOL, 8, _IDX_MINOR), lambda i: (i, 0, 0, 0)),
            pl.BlockSpec((_TCOL, _IDX_MINOR, _EMB), lambda i: (0, 0, 0)),
            pl.BlockSpec((1, _EMB), lambda i: (0, 0)),
            pl.BlockSpec((_EMB, 1), lambda i: (0, 0)),
            pl.BlockSpec((1, 1), lambda i: (0, 0)),
        ],
        out_specs=pl.BlockSpec((rows, 1), lambda i: (i, 0)),
        out_shape=jax.ShapeDtypeStruct((_BATCH, 1), jnp.float32),
    )(x4, W1p, b1.reshape(1, _EMB), W2, b2.reshape(1, 1))


def kernel(indices, table, W1, b1, W2, b2):
    idx = indices.astype(jnp.int32)
    idxp = jnp.concatenate(
        [idx, jnp.zeros((_BATCH, _SEQP - _SEQ), jnp.int32)], axis=1)
    idx5 = (idxp.reshape(_NW, 16, 8, _TCOL, 4)
            .transpose(0, 1, 3, 2, 4)
            .reshape(_NW, _GRP_PER_W, _IDX_MINOR))
    gathered = _sc_gather(idx5, table)
    x4 = gathered.reshape(_BATCH // 8, _TCOL, 8, _IDX_MINOR)
    W1p = jnp.concatenate(
        [W1, jnp.zeros((_TCOL * _IDX_MINOR - _SEQ * _EMB, _EMB), W1.dtype)],
        axis=0).reshape(_TCOL, _IDX_MINOR, _EMB)
    return _mlp(x4, W1p, b1, W2, b2)


# speedup vs baseline: 1.1943x; 1.1943x over previous
import functools

import jax
import jax.numpy as jnp
from jax import lax
from jax.experimental import pallas as pl
from jax.experimental.pallas import tpu as pltpu
from jax.experimental.pallas import tpu_sc as plsc

_BATCH = 4096
_SEQ = 50
_EMB = 32
_SEQP = 52
_TCOL = 13
_TOTAL = _BATCH * _SEQP

_NC = 2
_NS = 16
_NW = _NC * _NS
_ROWS_PER_W = _TOTAL // _NW
_IDX_MINOR = 128
_GRP_PER_W = _ROWS_PER_W // _IDX_MINOR
_GRP_PER_CHUNK = 13
_N_CHUNKS = _GRP_PER_W // _GRP_PER_CHUNK
_CHUNK_ROWS = _GRP_PER_CHUNK * _IDX_MINOR


def _sc_gather(idx3d, table):
    mesh = plsc.VectorSubcoreMesh(core_axis_name="c", subcore_axis_name="s")

    @functools.partial(
        pl.kernel,
        mesh=mesh,
        out_type=jax.ShapeDtypeStruct((_TOTAL, _EMB), jnp.float32),
        scratch_types=[
            pltpu.VMEM((_GRP_PER_W, _IDX_MINOR), jnp.int32),
            pltpu.VMEM((_CHUNK_ROWS, _EMB), jnp.float32),
            pltpu.SemaphoreType.DMA,
        ],
        compiler_params=pltpu.CompilerParams(use_tc_tiling_on_sc=False),
    )
    def gather_kernel(idx_hbm, table_hbm, out_hbm, idx_v, rows_v, sem):
        wid = lax.axis_index("s") * _NC + lax.axis_index("c")
        row_base = wid * _ROWS_PER_W
        pltpu.sync_copy(idx_hbm.at[wid], idx_v)
        for c in range(_N_CHUNKS):
            copies = []
            for j in range(_GRP_PER_CHUNK):
                copies.append(pltpu.async_copy(
                    table_hbm.at[idx_v.at[c * _GRP_PER_CHUNK + j]],
                    rows_v.at[pl.ds(j * _IDX_MINOR, _IDX_MINOR)],
                    sem,
                ))
            for cp in copies:
                cp.wait()
            pltpu.sync_copy(
                rows_v,
                out_hbm.at[pl.ds(row_base + c * _CHUNK_ROWS, _CHUNK_ROWS)],
            )

    return gather_kernel(idx3d, table)


def _mlp(x4, W1p, b1, W2, b2):
    gblk = 64
    rows = gblk * 8

    def body(x_ref, w1_ref, b1_ref, w2_ref, b2_ref, o_ref):
        acc = None
        for t in range(_TCOL):
            xt = x_ref[:, t].reshape(rows, _IDX_MINOR)
            p = jnp.dot(xt, w1_ref[t], preferred_element_type=jnp.float32)
            acc = p if acc is None else acc + p
        h = jnp.maximum(acc + b1_ref[...], 0.0)
        o = jnp.dot(h, w2_ref[...], preferred_element_type=jnp.float32)
        o_ref[...] = jnp.maximum(o + b2_ref[...], 0.0)

    return pl.pallas_call(
        body,
        grid=(_BATCH // rows,),
        in_specs=[
            pl.BlockSpec((gblk, _TCOL, 8, _IDX_MINOR), lambda i: (i, 0, 0, 0)),
            pl.BlockSpec((_TCOL, _IDX_MINOR, _EMB), lambda i: (0, 0, 0)),
            pl.BlockSpec((1, _EMB), lambda i: (0, 0)),
            pl.BlockSpec((_EMB, 1), lambda i: (0, 0)),
            pl.BlockSpec((1, 1), lambda i: (0, 0)),
        ],
        out_specs=pl.BlockSpec((rows, 1), lambda i: (i, 0)),
        out_shape=jax.ShapeDtypeStruct((_BATCH, 1), jnp.float32),
    )(x4, W1p, b1.reshape(1, _EMB), W2, b2.reshape(1, 1))


def kernel(indices, table, W1, b1, W2, b2):
    idx = indices.astype(jnp.int32)
    idxp = jnp.concatenate([idx, idx[:, : _SEQP - _SEQ]], axis=1)
    idx5 = (idxp.reshape(_NW, 16, 8, _TCOL, 4)
            .transpose(0, 1, 3, 2, 4)
            .reshape(_NW, _GRP_PER_W, _IDX_MINOR))
    gathered = _sc_gather(idx5, table)
    x4 = gathered.reshape(_BATCH // 8, _TCOL, 8, _IDX_MINOR)
    W1p = jnp.concatenate(
        [W1, jnp.zeros((_TCOL * _IDX_MINOR - _SEQ * _EMB, _EMB), W1.dtype)],
        axis=0).reshape(_TCOL, _IDX_MINOR, _EMB)
    return _mlp(x4, W1p, b1, W2, b2)
